# baseline (speedup 1.0000x reference)
import functools

import jax
import jax.numpy as jnp
from jax import lax
from jax.experimental import pallas as pl
from jax.experimental.pallas import tpu as pltpu
from jax.experimental.pallas import tpu_sc as plsc

NW = 32
CH = 128
N_BUF = 5


@functools.partial(jax.jit, static_argnames=("b", "d"))
def _gather_rows(table, idx_flat, b, d):
    b_per_w = b // NW
    n_steps = b_per_w // CH
    n_outer = n_steps // N_BUF

    mesh = plsc.VectorSubcoreMesh(core_axis_name="c", subcore_axis_name="s")

    @functools.partial(
        pl.kernel,
        out_type=jax.ShapeDtypeStruct((b, d), jnp.float32),
        mesh=mesh,
        scratch_types=[
            pltpu.VMEM((b_per_w,), jnp.int32),
            pltpu.VMEM_SHARED((256, d), jnp.float32),
            pltpu.VMEM((N_BUF, CH, d), jnp.float32),
            pltpu.SemaphoreType.DMA((N_BUF,)),
            pltpu.SemaphoreType.DMA((N_BUF,)),
        ],
    )
    def k(table_hbm, idx_hbm, out_hbm, idx_v, table_v, rows_v, gsem, wsem):
        wid = lax.axis_index("s") * 2 + lax.axis_index("c")
        base = wid * b_per_w
        sid = lax.axis_index("s")

        @pl.when(sid == 0)
        def _():
            pltpu.sync_copy(table_hbm, table_v.at[pl.ds(0, 201)])

        plsc.subcore_barrier()
        pltpu.sync_copy(idx_hbm.at[pl.ds(base, b_per_w)], idx_v)

        def start_gather(j, bf):
            pltpu.async_copy(
                table_v.at[idx_v.at[pl.ds(j * CH, CH)]],
                rows_v.at[bf], gsem.at[bf])

        def wait_gather(j, bf):
            pltpu.make_async_copy(
                table_v.at[idx_v.at[pl.ds(j * CH, CH)]],
                rows_v.at[bf], gsem.at[bf]).wait()

        def start_write(j, bf):
            pltpu.async_copy(
                rows_v.at[bf],
                out_hbm.at[pl.ds(base + j * CH, CH)], wsem.at[bf])

        def wait_write(j, bf):
            pltpu.make_async_copy(
                rows_v.at[bf],
                out_hbm.at[pl.ds(base + j * CH, CH)], wsem.at[bf]).wait()

        for bf in range(N_BUF):
            start_gather(bf, bf)

        def outer(g, carry):
            jb = g * N_BUF
            for bf in range(N_BUF):
                wait_gather(jb + bf, bf)
                start_write(jb + bf, bf)
            for bf in range(N_BUF):
                wait_write(jb + bf, bf)
                start_gather(jb + N_BUF + bf, bf)
            return carry

        lax.fori_loop(0, n_outer - 1, outer, 0)

        jb = (n_outer - 1) * N_BUF
        for bf in range(N_BUF):
            wait_gather(jb + bf, bf)
            start_write(jb + bf, bf)
        for bf in range(N_BUF):
            wait_write(jb + bf, bf)

    return k(table, idx_flat)


def kernel(x, posembedding_weight):
    b4, s = x.shape
    v, d = posembedding_weight.shape
    b = b4 * s
    idx_flat = x.reshape(b).astype(jnp.int32)
    out = _gather_rows(posembedding_weight, idx_flat, b, d)
    return out.reshape(b4, s, d)


# speedup vs baseline: 1.0138x; 1.0138x over previous
import functools

import jax
import jax.numpy as jnp
from jax import lax
from jax.experimental import pallas as pl
from jax.experimental.pallas import tpu as pltpu
from jax.experimental.pallas import tpu_sc as plsc

NW = 32
CH = 64
N_BUF = 8


@functools.partial(jax.jit, static_argnames=("b", "d"))
def _gather_rows(table, idx_flat, b, d):
    b_per_w = b // NW
    n_steps = b_per_w // CH
    n_outer = n_steps // N_BUF

    mesh = plsc.VectorSubcoreMesh(core_axis_name="c", subcore_axis_name="s")

    @functools.partial(
        pl.kernel,
        out_type=jax.ShapeDtypeStruct((b, d), jnp.float32),
        mesh=mesh,
        scratch_types=[
            pltpu.VMEM((b_per_w,), jnp.int32),
            pltpu.VMEM_SHARED((256, d), jnp.float32),
            pltpu.VMEM((N_BUF, CH, d), jnp.float32),
            pltpu.SemaphoreType.DMA((N_BUF,)),
            pltpu.SemaphoreType.DMA((N_BUF,)),
        ],
    )
    def k(table_hbm, idx_hbm, out_hbm, idx_v, table_v, rows_v, gsem, wsem):
        wid = lax.axis_index("s") * 2 + lax.axis_index("c")
        base = wid * b_per_w
        sid = lax.axis_index("s")

        @pl.when(sid == 0)
        def _():
            pltpu.sync_copy(table_hbm, table_v.at[pl.ds(0, 201)])

        plsc.subcore_barrier()
        pltpu.sync_copy(idx_hbm.at[pl.ds(base, b_per_w)], idx_v)

        def start_gather(j, bf):
            pltpu.async_copy(
                table_v.at[idx_v.at[pl.ds(j * CH, CH)]],
                rows_v.at[bf], gsem.at[bf])

        def wait_gather(j, bf):
            pltpu.make_async_copy(
                table_v.at[idx_v.at[pl.ds(j * CH, CH)]],
                rows_v.at[bf], gsem.at[bf]).wait()

        def start_write(j, bf):
            pltpu.async_copy(
                rows_v.at[bf],
                out_hbm.at[pl.ds(base + j * CH, CH)], wsem.at[bf])

        def wait_write(j, bf):
            pltpu.make_async_copy(
                rows_v.at[bf],
                out_hbm.at[pl.ds(base + j * CH, CH)], wsem.at[bf]).wait()

        for bf in range(N_BUF):
            start_gather(bf, bf)

        def outer(g, carry):
            jb = g * N_BUF
            for bf in range(N_BUF):
                wait_gather(jb + bf, bf)
                start_write(jb + bf, bf)
            for bf in range(N_BUF):
                wait_write(jb + bf, bf)
                start_gather(jb + N_BUF + bf, bf)
            return carry

        lax.fori_loop(0, n_outer - 1, outer, 0)

        jb = (n_outer - 1) * N_BUF
        for bf in range(N_BUF):
            wait_gather(jb + bf, bf)
            start_write(jb + bf, bf)
        for bf in range(N_BUF):
            wait_write(jb + bf, bf)

    return k(table, idx_flat)


def kernel(x, posembedding_weight):
    b4, s = x.shape
    v, d = posembedding_weight.shape
    b = b4 * s
    idx_flat = x.reshape(b).astype(jnp.int32)
    out = _gather_rows(posembedding_weight, idx_flat, b, d)
    return out.reshape(b4, s, d)


# speedup vs baseline: 1.0256x; 1.0116x over previous
import functools

import jax
import jax.numpy as jnp
from jax import lax
from jax.experimental import pallas as pl
from jax.experimental.pallas import tpu as pltpu
from jax.experimental.pallas import tpu_sc as plsc

NW = 32
CH = 64
N_BUF = 12


@functools.partial(jax.jit, static_argnames=("b", "d"))
def _gather_rows(table, idx_flat, b, d):
    b_per_w = b // NW
    n_steps = b_per_w // CH
    n_outer = n_steps // N_BUF

    mesh = plsc.VectorSubcoreMesh(core_axis_name="c", subcore_axis_name="s")

    @functools.partial(
        pl.kernel,
        out_type=jax.ShapeDtypeStruct((b, d), jnp.float32),
        mesh=mesh,
        scratch_types=[
            pltpu.VMEM((b_per_w,), jnp.int32),
            pltpu.VMEM_SHARED((256, d), jnp.float32),
            pltpu.VMEM((N_BUF, CH, d), jnp.float32),
            pltpu.SemaphoreType.DMA((N_BUF,)),
            pltpu.SemaphoreType.DMA((N_BUF,)),
        ],
    )
    def k(table_hbm, idx_hbm, out_hbm, idx_v, table_v, rows_v, gsem, wsem):
        wid = lax.axis_index("s") * 2 + lax.axis_index("c")
        base = wid * b_per_w
        sid = lax.axis_index("s")

        @pl.when(sid == 0)
        def _():
            pltpu.sync_copy(table_hbm, table_v.at[pl.ds(0, 201)])

        plsc.subcore_barrier()
        pltpu.sync_copy(idx_hbm.at[pl.ds(base, b_per_w)], idx_v)

        def start_gather(j, bf):
            pltpu.async_copy(
                table_v.at[idx_v.at[pl.ds(j * CH, CH)]],
                rows_v.at[bf], gsem.at[bf])

        def wait_gather(j, bf):
            pltpu.make_async_copy(
                table_v.at[idx_v.at[pl.ds(j * CH, CH)]],
                rows_v.at[bf], gsem.at[bf]).wait()

        def start_write(j, bf):
            pltpu.async_copy(
                rows_v.at[bf],
                out_hbm.at[pl.ds(base + j * CH, CH)], wsem.at[bf])

        def wait_write(j, bf):
            pltpu.make_async_copy(
                rows_v.at[bf],
                out_hbm.at[pl.ds(base + j * CH, CH)], wsem.at[bf]).wait()

        for bf in range(N_BUF):
            start_gather(bf, bf)

        def outer(g, carry):
            jb = g * N_BUF
            for bf in range(N_BUF):
                wait_gather(jb + bf, bf)
                start_write(jb + bf, bf)
            for bf in range(N_BUF):
                wait_write(jb + bf, bf)
                start_gather(jb + N_BUF + bf, bf)
            return carry

        lax.fori_loop(0, n_outer - 1, outer, 0)

        jb = (n_outer - 1) * N_BUF
        for bf in range(N_BUF):
            wait_gather(jb + bf, bf)
            start_write(jb + bf, bf)
        for bf in range(N_BUF):
            wait_write(jb + bf, bf)

    return k(table, idx_flat)


def kernel(x, posembedding_weight):
    b4, s = x.shape
    v, d = posembedding_weight.shape
    b = b4 * s
    idx_flat = x.reshape(b).astype(jnp.int32)
    out = _gather_rows(posembedding_weight, idx_flat, b, d)
    return out.reshape(b4, s, d)
